# Initial kernel scaffold; baseline (speedup 1.0000x reference)
import jax
import jax.numpy as jnp
from jax.experimental import pallas as pl


def kernel(input_p, input_x, params):
    raise NotImplementedError("write your pallas kernel here")


# speedup vs baseline: 12.3083x; 12.3083x over previous
import functools

import jax
import jax.numpy as jnp
from jax import lax
from jax.experimental import pallas as pl
from jax.experimental.pallas import tpu as pltpu
from jax.experimental.pallas import tpu_sc as plsc

D = 64
KNB = 16
PD = 16
EPS = 1e-5

TR = 128
TN = 256
TRK = TN * KNB

NC = 2
NS = 16
NW = NC * NS
CH = 128


def _knn_body(prow_ref, pcol_ref, idx_ref, *, n):
    pr = prow_ref[0]
    pc = pcol_ref[0]
    sq_r = jnp.zeros((TR, 1), jnp.float32)
    sq_c = jnp.zeros((1, n), jnp.float32)
    dot = jnp.zeros((TR, n), jnp.float32)
    for d in range(3):
        prd = pr[:, d:d + 1]
        pcd = pc[d:d + 1, :]
        sq_r = sq_r + prd * prd
        sq_c = sq_c + pcd * pcd
        prb = prd.astype(jnp.bfloat16).astype(jnp.float32)
        pcb = pcd.astype(jnp.bfloat16).astype(jnp.float32)
        dot = dot + prb * pcb
    acc = sq_r + sq_c - 2.0 * dot
    lanes = lax.broadcasted_iota(jnp.int32, (TR, n), 1)
    off = pl.program_id(0) * n
    for t in range(KNB):
        m = jnp.min(acc, axis=1, keepdims=True)
        am = jnp.min(jnp.where(acc == m, lanes, n), axis=1, keepdims=True)
        idx_ref[0, :, t:t + 1] = am + off
        acc = jnp.where(lanes == am, jnp.float32(jnp.inf), acc)


def _knn(p_rows, p_cols):
    b, n, _ = p_rows.shape
    return pl.pallas_call(
        functools.partial(_knn_body, n=n),
        grid=(b, n // TR),
        in_specs=[
            pl.BlockSpec((1, TR, 3), lambda bb, i: (bb, i, 0)),
            pl.BlockSpec((1, 3, n), lambda bb, i: (bb, 0, 0)),
        ],
        out_specs=pl.BlockSpec((1, TR, KNB), lambda bb, i: (bb, i, 0)),
        out_shape=jax.ShapeDtypeStruct((b, n, KNB), jnp.int32),
    )(p_rows, p_cols)


def _sc_gather(feat_tab, p_tab, idx_flat):
    rk = idx_flat.shape[0]
    per_w = rk // NW
    mesh = plsc.VectorSubcoreMesh(core_axis_name="c", subcore_axis_name="s")

    @functools.partial(
        pl.kernel,
        mesh=mesh,
        compiler_params=pltpu.CompilerParams(use_tc_tiling_on_sc=False),
        out_type=[
            jax.ShapeDtypeStruct((rk, 128), jnp.float32),
            jax.ShapeDtypeStruct((rk, PD), jnp.float32),
        ],
        scratch_types=[
            pltpu.VMEM((CH,), jnp.int32),
            pltpu.VMEM((CH, 128), jnp.float32),
            pltpu.VMEM((CH, PD), jnp.float32),
            pltpu.SemaphoreType.DMA,
            pltpu.SemaphoreType.DMA,
        ],
    )
    def k(feat_hbm, ptab_hbm, idx_hbm, fout_hbm, pout_hbm,
          idx_v, rows_v, prow_v, sem1, sem2):
        wid = lax.axis_index("s") * NC + lax.axis_index("c")
        base = wid * per_w

        def body(i, carry):
            off = base + i * CH
            pltpu.sync_copy(idx_hbm.at[pl.ds(off, CH)], idx_v)
            cp1 = pltpu.async_copy(feat_hbm.at[idx_v], rows_v, sem1)
            cp2 = pltpu.async_copy(ptab_hbm.at[idx_v], prow_v, sem2)
            cp1.wait()
            cp2.wait()
            pltpu.sync_copy(rows_v, fout_hbm.at[pl.ds(off, CH)])
            pltpu.sync_copy(prow_v, pout_hbm.at[pl.ds(off, CH)])
            return carry

        lax.fori_loop(0, per_w // CH, body, 0)

    return k(feat_tab, p_tab, idx_flat)


def _first(g):
    return g == 0


def _acc_stats(stats_ref, vals):
    s = jnp.sum(vals, axis=0, keepdims=True)
    ss = jnp.sum(vals * vals, axis=0, keepdims=True)

    @pl.when(_first(pl.program_id(0)))
    def _():
        stats_ref[...] = jnp.zeros_like(stats_ref)

    stats_ref[...] += jnp.concatenate([s, ss], axis=0)


def _top_body(x_ref, wt_ref, b_ref, t_ref, stats_ref):
    t = jnp.dot(x_ref[...], wt_ref[...],
                preferred_element_type=jnp.float32) + b_ref[...]
    t_ref[...] = t
    _acc_stats(stats_ref, t)


def _qkv_body(t_ref, aphi_ref, cphi_ref, aqk_ref, cqk_ref, phi_ref, qk_ref):
    t = t_ref[...]
    phi_ref[...] = jnp.dot(t, aphi_ref[...],
                           preferred_element_type=jnp.float32) + cphi_ref[...]
    qk_ref[...] = jnp.dot(t, aqk_ref[...],
                          preferred_element_type=jnp.float32) + cqk_ref[...]


def _rep_rows(x):
    c = x.shape[-1]
    x3 = jnp.broadcast_to(x[:, None, :], (TN, KNB, c))
    return x3.reshape(TN * KNB, c)


def _rel(ppad_ref, pg_ref):
    return _rep_rows(ppad_ref[...]) - pg_ref[...]


def _relstats_body(ppad_ref, pg_ref, stats_ref):
    rel = _rel(ppad_ref, pg_ref)
    s = jnp.sum(rel, axis=0, keepdims=True)
    m2 = lax.dot_general(rel, rel, (((0,), (0,)), ((), ())),
                         preferred_element_type=jnp.float32)

    @pl.when(_first(pl.program_id(0)))
    def _():
        stats_ref[...] = jnp.zeros_like(stats_ref)

    stats_ref[...] += jnp.concatenate([s, m2], axis=0)


def _v_body(ppad_ref, pg_ref, w1_ref, c1_ref, w2_ref, b2_ref, v_ref, stats_ref):
    rel = _rel(ppad_ref, pg_ref)
    pe1 = jnp.maximum(
        jnp.dot(rel, w1_ref[...], preferred_element_type=jnp.float32)
        + c1_ref[...], 0.0)
    v = jnp.dot(pe1, w2_ref[...], preferred_element_type=jnp.float32) + b2_ref[...]
    v_ref[...] = v
    _acc_stats(stats_ref, v)


def _a1_body(phi_ref, pag_ref, v_ref, av_ref, cv_ref, wg1_ref, bg1_ref,
             a1_ref, stats_ref):
    pe = v_ref[...] * av_ref[...] + cv_ref[...]
    attn_in = _rep_rows(phi_ref[...]) - pag_ref[:, :D] + pe
    a1 = jnp.dot(attn_in, wg1_ref[...],
                 preferred_element_type=jnp.float32) + bg1_ref[...]
    a1_ref[...] = a1
    _acc_stats(stats_ref, a1)


def _w2_body(a1_ref, aa_ref, ca_ref, wg2_ref, bg2_ref, w2_ref, stats_ref):
    a2 = jnp.maximum(a1_ref[...] * aa_ref[...] + ca_ref[...], 0.0)
    w2 = jnp.dot(a2, wg2_ref[...],
                 preferred_element_type=jnp.float32) + bg2_ref[...]
    w2_ref[...] = w2
    _acc_stats(stats_ref, w2)


def _out_body(w2_ref, sc_ref, pag_ref, v_ref, av_ref, cv_ref,
              wd_ref, bd_ref, y2_ref, stats_ref):
    logits = (w2_ref[...] * sc_ref[...]).reshape(TN, KNB, D)
    m = jnp.max(logits, axis=1, keepdims=True)
    e = jnp.exp(logits - m)
    denom = jnp.sum(e, axis=1)
    val = (pag_ref[:, D:] + v_ref[...] * av_ref[...] + cv_ref[...]
           ).reshape(TN, KNB, D)
    y = jnp.sum(e * val, axis=1) / denom
    y2 = jnp.dot(y, wd_ref[...], preferred_element_type=jnp.float32) + bd_ref[...]
    y2_ref[...] = y2
    _acc_stats(stats_ref, y2)


def _res_body(y2_ref, ay_ref, cy_ref, x0_ref, out_ref):
    out_ref[...] = y2_ref[...] * ay_ref[...] + cy_ref[...] + x0_ref[...]


def _full(shape):
    return pl.BlockSpec(shape, lambda g: tuple(0 for _ in shape))


def _rows(tile, c):
    return pl.BlockSpec((tile, c), lambda g: (g, 0))


def _call(body, grid, in_specs, ins, out_specs, out_shapes):
    return pl.pallas_call(
        body, grid=grid, in_specs=in_specs,
        out_specs=out_specs, out_shape=out_shapes)(*ins)


def _fold_bn(mean, var, gamma, beta):
    inv = gamma / jnp.sqrt(var + EPS)
    return inv, beta - mean * inv


def _stats_to_musig(stats, m):
    mu = stats[0] / m
    var = stats[1] / m - mu * mu
    return mu, var


def kernel(input_p, input_x, params):
    b, _, n = input_p.shape
    r = b * n
    rk = r * KNB
    grid_pts = (r // TN,)

    p_rows = jnp.transpose(input_p, (0, 2, 1))
    x0 = jnp.transpose(input_x, (0, 2, 1)).reshape(r, D)
    p_pad = jnp.concatenate(
        [p_rows, jnp.zeros((b, n, PD - 3), jnp.float32)], axis=-1
    ).reshape(r, PD)

    idx = _knn(p_rows, input_p)
    idx_flat = idx.reshape(rk)

    t_full, t_stats = _call(
        _top_body, grid_pts,
        [_rows(TN, D), _full((D, D)), _full((1, D))],
        [x0, params['W_top'].T, params['b_top'][None, :]],
        [_rows(TN, D), _full((2, D))],
        [jax.ShapeDtypeStruct((r, D), jnp.float32),
         jax.ShapeDtypeStruct((2, D), jnp.float32)],
    )
    mu_t, var_t = _stats_to_musig(t_stats, r)
    inv_t, sh_t = _fold_bn(mu_t, var_t, params['g_top'], params['be_top'])
    def _fold_conv(w, bb):
        return inv_t[:, None] * w.T, (sh_t @ w.T + bb)[None, :]
    a_phi, c_phi = _fold_conv(params['W_phi'], params['b_phi'])
    a_psi, c_psi = _fold_conv(params['W_psi'], params['b_psi'])
    a_al, c_al = _fold_conv(params['W_alpha'], params['b_alpha'])
    a_qk = jnp.concatenate([a_psi, a_al], axis=1)
    c_qk = jnp.concatenate([c_psi, c_al], axis=1)

    phi, qk = _call(
        _qkv_body, grid_pts,
        [_rows(TN, D), _full((D, D)), _full((1, D)),
         _full((D, 2 * D)), _full((1, 2 * D))],
        [t_full, a_phi, c_phi, a_qk, c_qk],
        [_rows(TN, D), _rows(TN, 2 * D)],
        [jax.ShapeDtypeStruct((r, D), jnp.float32),
         jax.ShapeDtypeStruct((r, 2 * D), jnp.float32)],
    )

    pa_g, p_g = _sc_gather(qk, p_pad, idx_flat)

    (rel_stats,) = _call(
        _relstats_body, grid_pts,
        [_rows(TN, PD), _rows(TRK, PD)],
        [p_pad, p_g],
        [_full((PD + 1, PD))],
        [jax.ShapeDtypeStruct((PD + 1, PD), jnp.float32)],
    )
    s_rel = rel_stats[0] / rk
    m2_rel = rel_stats[1:] / rk
    w_d1p = jnp.concatenate(
        [params['W_d1'], jnp.zeros((D, PD - 3), jnp.float32)], axis=1)
    mu_u = w_d1p @ s_rel + params['b_d1']
    e2_u = (jnp.einsum('oc,cd,od->o', w_d1p, m2_rel, w_d1p)
            + 2.0 * (w_d1p @ s_rel) * params['b_d1'] + params['b_d1'] ** 2)
    inv_u, sh_u = _fold_bn(mu_u, e2_u - mu_u * mu_u,
                           params['g_d1'], params['be_d1'])
    w1_hat = w_d1p.T * inv_u[None, :]
    c1_hat = (params['b_d1'] * inv_u + sh_u)[None, :]

    v_full, v_stats = _call(
        _v_body, grid_pts,
        [_rows(TN, PD), _rows(TRK, PD), _full((PD, D)), _full((1, D)),
         _full((D, D)), _full((1, D))],
        [p_pad, p_g, w1_hat, c1_hat, params['W_d2'].T,
         params['b_d2'][None, :]],
        [_rows(TRK, D), _full((2, D))],
        [jax.ShapeDtypeStruct((rk, D), jnp.float32),
         jax.ShapeDtypeStruct((2, D), jnp.float32)],
    )
    mu_v, var_v = _stats_to_musig(v_stats, rk)
    a_v, c_v = _fold_bn(mu_v, var_v, params['g_d2'], params['be_d2'])
    a_v, c_v = a_v[None, :], c_v[None, :]

    a1_full, a1_stats = _call(
        _a1_body, grid_pts,
        [_rows(TN, D), _rows(TRK, 2 * D), _rows(TRK, D), _full((1, D)),
         _full((1, D)), _full((D, D)), _full((1, D))],
        [phi, pa_g, v_full, a_v, c_v, params['W_g1'].T,
         params['b_g1'][None, :]],
        [_rows(TRK, D), _full((2, D))],
        [jax.ShapeDtypeStruct((rk, D), jnp.float32),
         jax.ShapeDtypeStruct((2, D), jnp.float32)],
    )
    mu_a1, var_a1 = _stats_to_musig(a1_stats, rk)
    a_a1, c_a1 = _fold_bn(mu_a1, var_a1, params['g_g1'], params['be_g1'])
    a_a1, c_a1 = a_a1[None, :], c_a1[None, :]

    w2_full, w2_stats = _call(
        _w2_body, grid_pts,
        [_rows(TRK, D), _full((1, D)), _full((1, D)), _full((D, D)),
         _full((1, D))],
        [a1_full, a_a1, c_a1, params['W_g2'].T, params['b_g2'][None, :]],
        [_rows(TRK, D), _full((2, D))],
        [jax.ShapeDtypeStruct((rk, D), jnp.float32),
         jax.ShapeDtypeStruct((2, D), jnp.float32)],
    )
    _, var_w2 = _stats_to_musig(w2_stats, rk)
    scale = (params['g_g2'] / jnp.sqrt(var_w2 + EPS))[None, :]

    y2_full, y2_stats = _call(
        _out_body, grid_pts,
        [_rows(TRK, D), _full((1, D)), _rows(TRK, 2 * D), _rows(TRK, D),
         _full((1, D)), _full((1, D)), _full((D, D)), _full((1, D))],
        [w2_full, scale, pa_g, v_full, a_v, c_v, params['W_down'].T,
         params['b_down'][None, :]],
        [_rows(TN, D), _full((2, D))],
        [jax.ShapeDtypeStruct((r, D), jnp.float32),
         jax.ShapeDtypeStruct((2, D), jnp.float32)],
    )
    mu_y2, var_y2 = _stats_to_musig(y2_stats, r)
    a_y, c_y = _fold_bn(mu_y2, var_y2, params['g_down'], params['be_down'])

    (out_rows,) = _call(
        _res_body, grid_pts,
        [_rows(TN, D), _full((1, D)), _full((1, D)), _rows(TN, D)],
        [y2_full, a_y[None, :], c_y[None, :], x0],
        [_rows(TN, D)],
        [jax.ShapeDtypeStruct((r, D), jnp.float32)],
    )
    return jnp.transpose(out_rows.reshape(b, n, D), (0, 2, 1))


# speedup vs baseline: 14.5649x; 1.1833x over previous
import functools

import jax
import jax.numpy as jnp
from jax import lax
from jax.experimental import pallas as pl
from jax.experimental.pallas import tpu as pltpu
from jax.experimental.pallas import tpu_sc as plsc

D = 64
KNB = 16
PD = 16
EPS = 1e-5

TR = 128
TN = 256
TRK = TN * KNB

NC = 2
NS = 16
NW = NC * NS
CH = 128


def _knn_body(prow_ref, pcol_ref, idx_ref, *, n):
    pr = prow_ref[0]
    pc = pcol_ref[0]
    sq_r = jnp.zeros((TR, 1), jnp.float32)
    sq_c = jnp.zeros((1, n), jnp.float32)
    dot = jnp.zeros((TR, n), jnp.float32)
    for d in range(3):
        prd = pr[:, d:d + 1]
        pcd = pc[d:d + 1, :]
        sq_r = sq_r + prd * prd
        sq_c = sq_c + pcd * pcd
        prb = prd.astype(jnp.bfloat16).astype(jnp.float32)
        pcb = pcd.astype(jnp.bfloat16).astype(jnp.float32)
        dot = dot + prb * pcb
    acc = sq_r + sq_c - 2.0 * dot
    lanes = lax.broadcasted_iota(jnp.int32, (TR, n), 1)
    off = pl.program_id(0) * n
    for t in range(KNB):
        am = jnp.argmin(acc, axis=1).astype(jnp.int32)[:, None]
        idx_ref[0, :, t:t + 1] = am + off
        acc = jnp.where(lanes == am, jnp.float32(jnp.inf), acc)


def _knn(p_rows, p_cols):
    b, n, _ = p_rows.shape
    return pl.pallas_call(
        functools.partial(_knn_body, n=n),
        grid=(b, n // TR),
        in_specs=[
            pl.BlockSpec((1, TR, 3), lambda bb, i: (bb, i, 0)),
            pl.BlockSpec((1, 3, n), lambda bb, i: (bb, 0, 0)),
        ],
        out_specs=pl.BlockSpec((1, TR, KNB), lambda bb, i: (bb, i, 0)),
        out_shape=jax.ShapeDtypeStruct((b, n, KNB), jnp.int32),
    )(p_rows, p_cols)


def _sc_gather(tab, idx_flat):
    rk = idx_flat.shape[0]
    c = tab.shape[1]
    per_w = rk // NW
    mesh = plsc.VectorSubcoreMesh(core_axis_name="c", subcore_axis_name="s")

    @functools.partial(
        pl.kernel,
        mesh=mesh,
        compiler_params=pltpu.CompilerParams(use_tc_tiling_on_sc=False),
        out_type=jax.ShapeDtypeStruct((rk, c), jnp.float32),
        scratch_types=[
            pltpu.VMEM((CH,), jnp.int32),
            pltpu.VMEM((CH, c), jnp.float32),
            pltpu.SemaphoreType.DMA,
        ],
    )
    def k(tab_hbm, idx_hbm, out_hbm, idx_v, rows_v, sem):
        wid = lax.axis_index("s") * NC + lax.axis_index("c")
        base = wid * per_w

        def body(i, carry):
            off = base + i * CH
            pltpu.sync_copy(idx_hbm.at[pl.ds(off, CH)], idx_v)
            pltpu.async_copy(tab_hbm.at[idx_v], rows_v, sem).wait()
            pltpu.sync_copy(rows_v, out_hbm.at[pl.ds(off, CH)])
            return carry

        lax.fori_loop(0, per_w // CH, body, 0)

    return k(tab, idx_flat)


def _first(g):
    return g == 0


def _acc_stats(stats_ref, vals):
    s = jnp.sum(vals, axis=0, keepdims=True)
    ss = jnp.sum(vals * vals, axis=0, keepdims=True)

    @pl.when(_first(pl.program_id(0)))
    def _():
        stats_ref[...] = jnp.zeros_like(stats_ref)

    stats_ref[...] += jnp.concatenate([s, ss], axis=0)


def _top_body(x_ref, wt_ref, b_ref, t_ref, stats_ref):
    t = jnp.dot(x_ref[...], wt_ref[...],
                preferred_element_type=jnp.float32) + b_ref[...]
    t_ref[...] = t
    _acc_stats(stats_ref, t)


def _qkv_body(t_ref, aphi_ref, cphi_ref, aqk_ref, cqk_ref, phi_ref, qk_ref):
    t = t_ref[...]
    phi_ref[...] = jnp.dot(t, aphi_ref[...],
                           preferred_element_type=jnp.float32) + cphi_ref[...]
    qk_ref[...] = jnp.dot(t, aqk_ref[...],
                          preferred_element_type=jnp.float32) + cqk_ref[...]


def _rep_rows(x):
    c = x.shape[-1]
    x3 = jnp.broadcast_to(x[:, None, :], (TN, KNB, c))
    return x3.reshape(TN * KNB, c)


def _rel(ppad_ref, pg_ref):
    return _rep_rows(ppad_ref[...]) - pg_ref[...]


def _relstats_body(ppad_ref, pg_ref, stats_ref):
    rel = _rel(ppad_ref, pg_ref)
    s = jnp.sum(rel, axis=0, keepdims=True)
    m2 = lax.dot_general(rel, rel, (((0,), (0,)), ((), ())),
                         preferred_element_type=jnp.float32)

    @pl.when(_first(pl.program_id(0)))
    def _():
        stats_ref[...] = jnp.zeros_like(stats_ref)

    stats_ref[...] += jnp.concatenate([s, m2], axis=0)


def _v_body(ppad_ref, pg_ref, w1_ref, c1_ref, w2_ref, b2_ref, v_ref, stats_ref):
    rel = _rel(ppad_ref, pg_ref)
    pe1 = jnp.maximum(
        jnp.dot(rel, w1_ref[...], preferred_element_type=jnp.float32)
        + c1_ref[...], 0.0)
    v = jnp.dot(pe1, w2_ref[...], preferred_element_type=jnp.float32) + b2_ref[...]
    v_ref[...] = v
    _acc_stats(stats_ref, v)


def _a1_body(phi_ref, pag_ref, v_ref, av_ref, cv_ref, wg1_ref, bg1_ref,
             a1_ref, stats_ref):
    pe = v_ref[...] * av_ref[...] + cv_ref[...]
    attn_in = _rep_rows(phi_ref[...]) - pag_ref[:, :D] + pe
    a1 = jnp.dot(attn_in, wg1_ref[...],
                 preferred_element_type=jnp.float32) + bg1_ref[...]
    a1_ref[...] = a1
    _acc_stats(stats_ref, a1)


def _w2_body(a1_ref, aa_ref, ca_ref, wg2_ref, bg2_ref, w2_ref, stats_ref):
    a2 = jnp.maximum(a1_ref[...] * aa_ref[...] + ca_ref[...], 0.0)
    w2 = jnp.dot(a2, wg2_ref[...],
                 preferred_element_type=jnp.float32) + bg2_ref[...]
    w2_ref[...] = w2
    _acc_stats(stats_ref, w2)


def _out_body(w2_ref, sc_ref, pag_ref, v_ref, av_ref, cv_ref,
              wd_ref, bd_ref, y2_ref, stats_ref):
    logits = (w2_ref[...] * sc_ref[...]).reshape(TN, KNB, D)
    m = jnp.max(logits, axis=1, keepdims=True)
    e = jnp.exp(logits - m)
    denom = jnp.sum(e, axis=1)
    val = (pag_ref[:, D:] + v_ref[...] * av_ref[...] + cv_ref[...]
           ).reshape(TN, KNB, D)
    y = jnp.sum(e * val, axis=1) / denom
    y2 = jnp.dot(y, wd_ref[...], preferred_element_type=jnp.float32) + bd_ref[...]
    y2_ref[...] = y2
    _acc_stats(stats_ref, y2)


def _res_body(y2_ref, ay_ref, cy_ref, x0_ref, out_ref):
    out_ref[...] = y2_ref[...] * ay_ref[...] + cy_ref[...] + x0_ref[...]


def _full(shape):
    return pl.BlockSpec(shape, lambda g: tuple(0 for _ in shape))


def _rows(tile, c):
    return pl.BlockSpec((tile, c), lambda g: (g, 0))


def _call(body, grid, in_specs, ins, out_specs, out_shapes):
    return pl.pallas_call(
        body, grid=grid, in_specs=in_specs,
        out_specs=out_specs, out_shape=out_shapes)(*ins)


def _fold_bn(mean, var, gamma, beta):
    inv = gamma / jnp.sqrt(var + EPS)
    return inv, beta - mean * inv


def _stats_to_musig(stats, m):
    mu = stats[0] / m
    var = stats[1] / m - mu * mu
    return mu, var


def kernel(input_p, input_x, params):
    b, _, n = input_p.shape
    r = b * n
    rk = r * KNB
    grid_pts = (r // TN,)

    p_rows = jnp.transpose(input_p, (0, 2, 1))
    x0 = jnp.transpose(input_x, (0, 2, 1)).reshape(r, D)
    p_pad = jnp.concatenate(
        [p_rows, jnp.zeros((b, n, PD - 3), jnp.float32)], axis=-1
    ).reshape(r, PD)

    idx = _knn(p_rows, input_p)
    idx_flat = idx.reshape(rk)

    p_g = _sc_gather(p_pad, idx_flat)

    t_full, t_stats = _call(
        _top_body, grid_pts,
        [_rows(TN, D), _full((D, D)), _full((1, D))],
        [x0, params['W_top'].T, params['b_top'][None, :]],
        [_rows(TN, D), _full((2, D))],
        [jax.ShapeDtypeStruct((r, D), jnp.float32),
         jax.ShapeDtypeStruct((2, D), jnp.float32)],
    )
    mu_t, var_t = _stats_to_musig(t_stats, r)
    inv_t, sh_t = _fold_bn(mu_t, var_t, params['g_top'], params['be_top'])
    def _fold_conv(w, bb):
        return inv_t[:, None] * w.T, (sh_t @ w.T + bb)[None, :]
    a_phi, c_phi = _fold_conv(params['W_phi'], params['b_phi'])
    a_psi, c_psi = _fold_conv(params['W_psi'], params['b_psi'])
    a_al, c_al = _fold_conv(params['W_alpha'], params['b_alpha'])
    a_qk = jnp.concatenate([a_psi, a_al], axis=1)
    c_qk = jnp.concatenate([c_psi, c_al], axis=1)

    phi, qk = _call(
        _qkv_body, grid_pts,
        [_rows(TN, D), _full((D, D)), _full((1, D)),
         _full((D, 2 * D)), _full((1, 2 * D))],
        [t_full, a_phi, c_phi, a_qk, c_qk],
        [_rows(TN, D), _rows(TN, 2 * D)],
        [jax.ShapeDtypeStruct((r, D), jnp.float32),
         jax.ShapeDtypeStruct((r, 2 * D), jnp.float32)],
    )

    pa_g = _sc_gather(qk, idx_flat)

    (rel_stats,) = _call(
        _relstats_body, grid_pts,
        [_rows(TN, PD), _rows(TRK, PD)],
        [p_pad, p_g],
        [_full((PD + 1, PD))],
        [jax.ShapeDtypeStruct((PD + 1, PD), jnp.float32)],
    )
    s_rel = rel_stats[0] / rk
    m2_rel = rel_stats[1:] / rk
    w_d1p = jnp.concatenate(
        [params['W_d1'], jnp.zeros((D, PD - 3), jnp.float32)], axis=1)
    mu_u = w_d1p @ s_rel + params['b_d1']
    e2_u = (jnp.einsum('oc,cd,od->o', w_d1p, m2_rel, w_d1p)
            + 2.0 * (w_d1p @ s_rel) * params['b_d1'] + params['b_d1'] ** 2)
    inv_u, sh_u = _fold_bn(mu_u, e2_u - mu_u * mu_u,
                           params['g_d1'], params['be_d1'])
    w1_hat = w_d1p.T * inv_u[None, :]
    c1_hat = (params['b_d1'] * inv_u + sh_u)[None, :]

    v_full, v_stats = _call(
        _v_body, grid_pts,
        [_rows(TN, PD), _rows(TRK, PD), _full((PD, D)), _full((1, D)),
         _full((D, D)), _full((1, D))],
        [p_pad, p_g, w1_hat, c1_hat, params['W_d2'].T,
         params['b_d2'][None, :]],
        [_rows(TRK, D), _full((2, D))],
        [jax.ShapeDtypeStruct((rk, D), jnp.float32),
         jax.ShapeDtypeStruct((2, D), jnp.float32)],
    )
    mu_v, var_v = _stats_to_musig(v_stats, rk)
    a_v, c_v = _fold_bn(mu_v, var_v, params['g_d2'], params['be_d2'])
    a_v, c_v = a_v[None, :], c_v[None, :]

    a1_full, a1_stats = _call(
        _a1_body, grid_pts,
        [_rows(TN, D), _rows(TRK, 2 * D), _rows(TRK, D), _full((1, D)),
         _full((1, D)), _full((D, D)), _full((1, D))],
        [phi, pa_g, v_full, a_v, c_v, params['W_g1'].T,
         params['b_g1'][None, :]],
        [_rows(TRK, D), _full((2, D))],
        [jax.ShapeDtypeStruct((rk, D), jnp.float32),
         jax.ShapeDtypeStruct((2, D), jnp.float32)],
    )
    mu_a1, var_a1 = _stats_to_musig(a1_stats, rk)
    a_a1, c_a1 = _fold_bn(mu_a1, var_a1, params['g_g1'], params['be_g1'])
    a_a1, c_a1 = a_a1[None, :], c_a1[None, :]

    w2_full, w2_stats = _call(
        _w2_body, grid_pts,
        [_rows(TRK, D), _full((1, D)), _full((1, D)), _full((D, D)),
         _full((1, D))],
        [a1_full, a_a1, c_a1, params['W_g2'].T, params['b_g2'][None, :]],
        [_rows(TRK, D), _full((2, D))],
        [jax.ShapeDtypeStruct((rk, D), jnp.float32),
         jax.ShapeDtypeStruct((2, D), jnp.float32)],
    )
    _, var_w2 = _stats_to_musig(w2_stats, rk)
    scale = (params['g_g2'] / jnp.sqrt(var_w2 + EPS))[None, :]

    y2_full, y2_stats = _call(
        _out_body, grid_pts,
        [_rows(TRK, D), _full((1, D)), _rows(TRK, 2 * D), _rows(TRK, D),
         _full((1, D)), _full((1, D)), _full((D, D)), _full((1, D))],
        [w2_full, scale, pa_g, v_full, a_v, c_v, params['W_down'].T,
         params['b_down'][None, :]],
        [_rows(TN, D), _full((2, D))],
        [jax.ShapeDtypeStruct((r, D), jnp.float32),
         jax.ShapeDtypeStruct((2, D), jnp.float32)],
    )
    mu_y2, var_y2 = _stats_to_musig(y2_stats, r)
    a_y, c_y = _fold_bn(mu_y2, var_y2, params['g_down'], params['be_down'])

    (out_rows,) = _call(
        _res_body, grid_pts,
        [_rows(TN, D), _full((1, D)), _full((1, D)), _rows(TN, D)],
        [y2_full, a_y[None, :], c_y[None, :], x0],
        [_rows(TN, D)],
        [jax.ShapeDtypeStruct((r, D), jnp.float32)],
    )
    return jnp.transpose(out_rows.reshape(b, n, D), (0, 2, 1))


# speedup vs baseline: 14.7922x; 1.0156x over previous
import functools

import jax
import jax.numpy as jnp
from jax import lax
from jax.experimental import pallas as pl
from jax.experimental.pallas import tpu as pltpu
from jax.experimental.pallas import tpu_sc as plsc

D = 64
KNB = 16
PD = 16
EPS = 1e-5

TR = 256
TN = 256
TRK = TN * KNB

NC = 2
NS = 16
NW = NC * NS
CH = 128


def _knn_body(prow_ref, pcol_ref, idx_ref, *, n):
    pr = prow_ref[0]
    pc = pcol_ref[0]
    sq_r = jnp.zeros((TR, 1), jnp.float32)
    sq_c = jnp.zeros((1, n), jnp.float32)
    dot = jnp.zeros((TR, n), jnp.float32)
    for d in range(3):
        prd = pr[:, d:d + 1]
        pcd = pc[d:d + 1, :]
        sq_r = sq_r + prd * prd
        sq_c = sq_c + pcd * pcd
        prb = prd.astype(jnp.bfloat16).astype(jnp.float32)
        pcb = pcd.astype(jnp.bfloat16).astype(jnp.float32)
        dot = dot + prb * pcb
    acc = sq_r + sq_c - 2.0 * dot
    lanes = lax.broadcasted_iota(jnp.int32, (TR, n), 1)
    off = pl.program_id(0) * n
    for t in range(KNB):
        am = jnp.argmin(acc, axis=1).astype(jnp.int32)[:, None]
        idx_ref[0, :, t:t + 1] = am + off
        acc = jnp.where(lanes == am, jnp.float32(jnp.inf), acc)


def _knn(p_rows, p_cols):
    b, n, _ = p_rows.shape
    return pl.pallas_call(
        functools.partial(_knn_body, n=n),
        grid=(b, n // TR),
        in_specs=[
            pl.BlockSpec((1, TR, 3), lambda bb, i: (bb, i, 0)),
            pl.BlockSpec((1, 3, n), lambda bb, i: (bb, 0, 0)),
        ],
        out_specs=pl.BlockSpec((1, TR, KNB), lambda bb, i: (bb, i, 0)),
        out_shape=jax.ShapeDtypeStruct((b, n, KNB), jnp.int32),
    )(p_rows, p_cols)


def _sc_gather(tab, idx_flat):
    rk = idx_flat.shape[0]
    c = tab.shape[1]
    per_w = rk // NW
    mesh = plsc.VectorSubcoreMesh(core_axis_name="c", subcore_axis_name="s")

    @functools.partial(
        pl.kernel,
        mesh=mesh,
        compiler_params=pltpu.CompilerParams(use_tc_tiling_on_sc=False),
        out_type=jax.ShapeDtypeStruct((rk, c), jnp.float32),
        scratch_types=[
            pltpu.VMEM((per_w,), jnp.int32),
            pltpu.VMEM((CH, c), jnp.float32),
            pltpu.VMEM((CH, c), jnp.float32),
            pltpu.SemaphoreType.DMA,
            pltpu.SemaphoreType.DMA,
            pltpu.SemaphoreType.DMA,
            pltpu.SemaphoreType.DMA,
        ],
    )
    def k(tab_hbm, idx_hbm, out_hbm, idx_all, rows_v0, rows_v1,
          semg0, semg1, semo0, semo1):
        wid = lax.axis_index("s") * NC + lax.axis_index("c")
        base = wid * per_w
        pltpu.sync_copy(idx_hbm.at[pl.ds(base, per_w)], idx_all)

        def body(i, carry):
            g0 = 2 * i
            g1 = 2 * i + 1
            cg0 = pltpu.async_copy(
                tab_hbm.at[idx_all.at[pl.ds(g0 * CH, CH)]], rows_v0, semg0)
            cg1 = pltpu.async_copy(
                tab_hbm.at[idx_all.at[pl.ds(g1 * CH, CH)]], rows_v1, semg1)
            cg0.wait()
            co0 = pltpu.async_copy(
                rows_v0, out_hbm.at[pl.ds(base + g0 * CH, CH)], semo0)
            cg1.wait()
            co1 = pltpu.async_copy(
                rows_v1, out_hbm.at[pl.ds(base + g1 * CH, CH)], semo1)
            co0.wait()
            co1.wait()
            return carry

        lax.fori_loop(0, per_w // (2 * CH), body, 0)

    return k(tab, idx_flat)


def _first(g):
    return g == 0


def _acc_stats(stats_ref, vals):
    s = jnp.sum(vals, axis=0, keepdims=True)
    ss = jnp.sum(vals * vals, axis=0, keepdims=True)

    @pl.when(_first(pl.program_id(0)))
    def _():
        stats_ref[...] = jnp.zeros_like(stats_ref)

    stats_ref[...] += jnp.concatenate([s, ss], axis=0)


def _top_body(x_ref, wt_ref, b_ref, t_ref, stats_ref):
    t = jnp.dot(x_ref[...], wt_ref[...],
                preferred_element_type=jnp.float32) + b_ref[...]
    t_ref[...] = t
    _acc_stats(stats_ref, t)


def _qkv_body(t_ref, aphi_ref, cphi_ref, aqk_ref, cqk_ref, phi_ref, qk_ref):
    t = t_ref[...]
    phi_ref[...] = jnp.dot(t, aphi_ref[...],
                           preferred_element_type=jnp.float32) + cphi_ref[...]
    qk_ref[...] = jnp.dot(t, aqk_ref[...],
                          preferred_element_type=jnp.float32) + cqk_ref[...]


def _rep_rows(x):
    c = x.shape[-1]
    x3 = jnp.broadcast_to(x[:, None, :], (TN, KNB, c))
    return x3.reshape(TN * KNB, c)


def _rel(ppad_ref, pg_ref):
    return _rep_rows(ppad_ref[...]) - pg_ref[...]


def _relstats_body(ppad_ref, pg_ref, stats_ref):
    rel = _rel(ppad_ref, pg_ref)
    s = jnp.sum(rel, axis=0, keepdims=True)
    m2 = lax.dot_general(rel, rel, (((0,), (0,)), ((), ())),
                         preferred_element_type=jnp.float32)

    @pl.when(_first(pl.program_id(0)))
    def _():
        stats_ref[...] = jnp.zeros_like(stats_ref)

    stats_ref[...] += jnp.concatenate([s, m2], axis=0)


def _v_body(ppad_ref, pg_ref, w1_ref, c1_ref, w2_ref, b2_ref, v_ref, stats_ref):
    rel = _rel(ppad_ref, pg_ref)
    pe1 = jnp.maximum(
        jnp.dot(rel, w1_ref[...], preferred_element_type=jnp.float32)
        + c1_ref[...], 0.0)
    v = jnp.dot(pe1, w2_ref[...], preferred_element_type=jnp.float32) + b2_ref[...]
    v_ref[...] = v
    _acc_stats(stats_ref, v)


def _a1_body(phi_ref, pag_ref, v_ref, av_ref, cv_ref, wg1_ref, bg1_ref,
             a1_ref, stats_ref):
    pe = v_ref[...] * av_ref[...] + cv_ref[...]
    attn_in = _rep_rows(phi_ref[...]) - pag_ref[:, :D] + pe
    a1 = jnp.dot(attn_in, wg1_ref[...],
                 preferred_element_type=jnp.float32) + bg1_ref[...]
    a1_ref[...] = a1
    _acc_stats(stats_ref, a1)


def _w2_body(a1_ref, aa_ref, ca_ref, wg2_ref, bg2_ref, w2_ref, stats_ref):
    a2 = jnp.maximum(a1_ref[...] * aa_ref[...] + ca_ref[...], 0.0)
    w2 = jnp.dot(a2, wg2_ref[...],
                 preferred_element_type=jnp.float32) + bg2_ref[...]
    w2_ref[...] = w2
    _acc_stats(stats_ref, w2)


def _out_body(w2_ref, sc_ref, pag_ref, v_ref, av_ref, cv_ref,
              wd_ref, bd_ref, y2_ref, stats_ref):
    logits = (w2_ref[...] * sc_ref[...]).reshape(TN, KNB, D)
    m = jnp.max(logits, axis=1, keepdims=True)
    e = jnp.exp(logits - m)
    denom = jnp.sum(e, axis=1)
    val = (pag_ref[:, D:] + v_ref[...] * av_ref[...] + cv_ref[...]
           ).reshape(TN, KNB, D)
    y = jnp.sum(e * val, axis=1) / denom
    y2 = jnp.dot(y, wd_ref[...], preferred_element_type=jnp.float32) + bd_ref[...]
    y2_ref[...] = y2
    _acc_stats(stats_ref, y2)


def _res_body(y2_ref, ay_ref, cy_ref, x0_ref, out_ref):
    out_ref[...] = y2_ref[...] * ay_ref[...] + cy_ref[...] + x0_ref[...]


def _full(shape):
    return pl.BlockSpec(shape, lambda g: tuple(0 for _ in shape))


def _rows(tile, c):
    return pl.BlockSpec((tile, c), lambda g: (g, 0))


def _call(body, grid, in_specs, ins, out_specs, out_shapes):
    return pl.pallas_call(
        body, grid=grid, in_specs=in_specs,
        out_specs=out_specs, out_shape=out_shapes)(*ins)


def _fold_bn(mean, var, gamma, beta):
    inv = gamma / jnp.sqrt(var + EPS)
    return inv, beta - mean * inv


def _stats_to_musig(stats, m):
    mu = stats[0] / m
    var = stats[1] / m - mu * mu
    return mu, var


def kernel(input_p, input_x, params):
    b, _, n = input_p.shape
    r = b * n
    rk = r * KNB
    grid_pts = (r // TN,)

    p_rows = jnp.transpose(input_p, (0, 2, 1))
    x0 = jnp.transpose(input_x, (0, 2, 1)).reshape(r, D)
    p_pad = jnp.concatenate(
        [p_rows, jnp.zeros((b, n, PD - 3), jnp.float32)], axis=-1
    ).reshape(r, PD)

    idx = _knn(p_rows, input_p)
    idx_flat = idx.reshape(rk)

    p_g = _sc_gather(p_pad, idx_flat)

    t_full, t_stats = _call(
        _top_body, grid_pts,
        [_rows(TN, D), _full((D, D)), _full((1, D))],
        [x0, params['W_top'].T, params['b_top'][None, :]],
        [_rows(TN, D), _full((2, D))],
        [jax.ShapeDtypeStruct((r, D), jnp.float32),
         jax.ShapeDtypeStruct((2, D), jnp.float32)],
    )
    mu_t, var_t = _stats_to_musig(t_stats, r)
    inv_t, sh_t = _fold_bn(mu_t, var_t, params['g_top'], params['be_top'])
    def _fold_conv(w, bb):
        return inv_t[:, None] * w.T, (sh_t @ w.T + bb)[None, :]
    a_phi, c_phi = _fold_conv(params['W_phi'], params['b_phi'])
    a_psi, c_psi = _fold_conv(params['W_psi'], params['b_psi'])
    a_al, c_al = _fold_conv(params['W_alpha'], params['b_alpha'])
    a_qk = jnp.concatenate([a_psi, a_al], axis=1)
    c_qk = jnp.concatenate([c_psi, c_al], axis=1)

    phi, qk = _call(
        _qkv_body, grid_pts,
        [_rows(TN, D), _full((D, D)), _full((1, D)),
         _full((D, 2 * D)), _full((1, 2 * D))],
        [t_full, a_phi, c_phi, a_qk, c_qk],
        [_rows(TN, D), _rows(TN, 2 * D)],
        [jax.ShapeDtypeStruct((r, D), jnp.float32),
         jax.ShapeDtypeStruct((r, 2 * D), jnp.float32)],
    )

    pa_g = _sc_gather(qk, idx_flat)

    (rel_stats,) = _call(
        _relstats_body, grid_pts,
        [_rows(TN, PD), _rows(TRK, PD)],
        [p_pad, p_g],
        [_full((PD + 1, PD))],
        [jax.ShapeDtypeStruct((PD + 1, PD), jnp.float32)],
    )
    s_rel = rel_stats[0] / rk
    m2_rel = rel_stats[1:] / rk
    w_d1p = jnp.concatenate(
        [params['W_d1'], jnp.zeros((D, PD - 3), jnp.float32)], axis=1)
    mu_u = w_d1p @ s_rel + params['b_d1']
    e2_u = (jnp.einsum('oc,cd,od->o', w_d1p, m2_rel, w_d1p)
            + 2.0 * (w_d1p @ s_rel) * params['b_d1'] + params['b_d1'] ** 2)
    inv_u, sh_u = _fold_bn(mu_u, e2_u - mu_u * mu_u,
                           params['g_d1'], params['be_d1'])
    w1_hat = w_d1p.T * inv_u[None, :]
    c1_hat = (params['b_d1'] * inv_u + sh_u)[None, :]

    v_full, v_stats = _call(
        _v_body, grid_pts,
        [_rows(TN, PD), _rows(TRK, PD), _full((PD, D)), _full((1, D)),
         _full((D, D)), _full((1, D))],
        [p_pad, p_g, w1_hat, c1_hat, params['W_d2'].T,
         params['b_d2'][None, :]],
        [_rows(TRK, D), _full((2, D))],
        [jax.ShapeDtypeStruct((rk, D), jnp.float32),
         jax.ShapeDtypeStruct((2, D), jnp.float32)],
    )
    mu_v, var_v = _stats_to_musig(v_stats, rk)
    a_v, c_v = _fold_bn(mu_v, var_v, params['g_d2'], params['be_d2'])
    a_v, c_v = a_v[None, :], c_v[None, :]

    a1_full, a1_stats = _call(
        _a1_body, grid_pts,
        [_rows(TN, D), _rows(TRK, 2 * D), _rows(TRK, D), _full((1, D)),
         _full((1, D)), _full((D, D)), _full((1, D))],
        [phi, pa_g, v_full, a_v, c_v, params['W_g1'].T,
         params['b_g1'][None, :]],
        [_rows(TRK, D), _full((2, D))],
        [jax.ShapeDtypeStruct((rk, D), jnp.float32),
         jax.ShapeDtypeStruct((2, D), jnp.float32)],
    )
    mu_a1, var_a1 = _stats_to_musig(a1_stats, rk)
    a_a1, c_a1 = _fold_bn(mu_a1, var_a1, params['g_g1'], params['be_g1'])
    a_a1, c_a1 = a_a1[None, :], c_a1[None, :]

    w2_full, w2_stats = _call(
        _w2_body, grid_pts,
        [_rows(TRK, D), _full((1, D)), _full((1, D)), _full((D, D)),
         _full((1, D))],
        [a1_full, a_a1, c_a1, params['W_g2'].T, params['b_g2'][None, :]],
        [_rows(TRK, D), _full((2, D))],
        [jax.ShapeDtypeStruct((rk, D), jnp.float32),
         jax.ShapeDtypeStruct((2, D), jnp.float32)],
    )
    _, var_w2 = _stats_to_musig(w2_stats, rk)
    scale = (params['g_g2'] / jnp.sqrt(var_w2 + EPS))[None, :]

    y2_full, y2_stats = _call(
        _out_body, grid_pts,
        [_rows(TRK, D), _full((1, D)), _rows(TRK, 2 * D), _rows(TRK, D),
         _full((1, D)), _full((1, D)), _full((D, D)), _full((1, D))],
        [w2_full, scale, pa_g, v_full, a_v, c_v, params['W_down'].T,
         params['b_down'][None, :]],
        [_rows(TN, D), _full((2, D))],
        [jax.ShapeDtypeStruct((r, D), jnp.float32),
         jax.ShapeDtypeStruct((2, D), jnp.float32)],
    )
    mu_y2, var_y2 = _stats_to_musig(y2_stats, r)
    a_y, c_y = _fold_bn(mu_y2, var_y2, params['g_down'], params['be_down'])

    (out_rows,) = _call(
        _res_body, grid_pts,
        [_rows(TN, D), _full((1, D)), _full((1, D)), _rows(TN, D)],
        [y2_full, a_y[None, :], c_y[None, :], x0],
        [_rows(TN, D)],
        [jax.ShapeDtypeStruct((r, D), jnp.float32)],
    )
    return jnp.transpose(out_rows.reshape(b, n, D), (0, 2, 1))


# speedup vs baseline: 15.7836x; 1.0670x over previous
import functools

import jax
import jax.numpy as jnp
from jax import lax
from jax.experimental import pallas as pl
from jax.experimental.pallas import tpu as pltpu
from jax.experimental.pallas import tpu_sc as plsc

D = 64
KNB = 16
PD = 16
EPS = 1e-5

TR = 256
TN = 512
TRK = TN * KNB

NC = 2
NS = 16
NW = NC * NS
CH = 128


def _knn_body(prow_ref, pcol_ref, idx_ref, *, n):
    pr = prow_ref[0]
    pc = pcol_ref[0]
    sq_r = jnp.zeros((TR, 1), jnp.float32)
    sq_c = jnp.zeros((1, n), jnp.float32)
    dot = jnp.zeros((TR, n), jnp.float32)
    for d in range(3):
        prd = pr[:, d:d + 1]
        pcd = pc[d:d + 1, :]
        sq_r = sq_r + prd * prd
        sq_c = sq_c + pcd * pcd
        prb = prd.astype(jnp.bfloat16).astype(jnp.float32)
        pcb = pcd.astype(jnp.bfloat16).astype(jnp.float32)
        dot = dot + prb * pcb
    acc = sq_r + sq_c - 2.0 * dot
    lanes = lax.broadcasted_iota(jnp.int32, (TR, n), 1)
    off = pl.program_id(0) * n
    for t in range(KNB):
        am = jnp.argmin(acc, axis=1).astype(jnp.int32)[:, None]
        idx_ref[0, :, t:t + 1] = am + off
        acc = jnp.where(lanes == am, jnp.float32(jnp.inf), acc)


def _knn(p_rows, p_cols):
    b, n, _ = p_rows.shape
    return pl.pallas_call(
        functools.partial(_knn_body, n=n),
        grid=(b, n // TR),
        in_specs=[
            pl.BlockSpec((1, TR, 3), lambda bb, i: (bb, i, 0)),
            pl.BlockSpec((1, 3, n), lambda bb, i: (bb, 0, 0)),
        ],
        out_specs=pl.BlockSpec((1, TR, KNB), lambda bb, i: (bb, i, 0)),
        out_shape=jax.ShapeDtypeStruct((b, n, KNB), jnp.int32),
    )(p_rows, p_cols)


def _sc_gather(tab, idx_flat):
    rk = idx_flat.shape[0]
    c = tab.shape[1]
    per_w = rk // NW
    mesh = plsc.VectorSubcoreMesh(core_axis_name="c", subcore_axis_name="s")

    @functools.partial(
        pl.kernel,
        mesh=mesh,
        compiler_params=pltpu.CompilerParams(use_tc_tiling_on_sc=False),
        out_type=jax.ShapeDtypeStruct((rk, c), jnp.float32),
        scratch_types=[
            pltpu.VMEM((per_w,), jnp.int32),
            pltpu.VMEM((CH, c), jnp.float32),
            pltpu.VMEM((CH, c), jnp.float32),
            pltpu.SemaphoreType.DMA,
            pltpu.SemaphoreType.DMA,
            pltpu.SemaphoreType.DMA,
            pltpu.SemaphoreType.DMA,
        ],
    )
    def k(tab_hbm, idx_hbm, out_hbm, idx_all, rows_v0, rows_v1,
          semg0, semg1, semo0, semo1):
        wid = lax.axis_index("s") * NC + lax.axis_index("c")
        base = wid * per_w
        pltpu.sync_copy(idx_hbm.at[pl.ds(base, per_w)], idx_all)

        def body(i, carry):
            g0 = 2 * i
            g1 = 2 * i + 1
            cg0 = pltpu.async_copy(
                tab_hbm.at[idx_all.at[pl.ds(g0 * CH, CH)]], rows_v0, semg0)
            cg1 = pltpu.async_copy(
                tab_hbm.at[idx_all.at[pl.ds(g1 * CH, CH)]], rows_v1, semg1)
            cg0.wait()
            co0 = pltpu.async_copy(
                rows_v0, out_hbm.at[pl.ds(base + g0 * CH, CH)], semo0)
            cg1.wait()
            co1 = pltpu.async_copy(
                rows_v1, out_hbm.at[pl.ds(base + g1 * CH, CH)], semo1)
            co0.wait()
            co1.wait()
            return carry

        lax.fori_loop(0, per_w // (2 * CH), body, 0)

    return k(tab, idx_flat)


def _first(g):
    return g == 0


def _acc_stats(stats_ref, vals):
    s = jnp.sum(vals, axis=0, keepdims=True)
    ss = jnp.sum(vals * vals, axis=0, keepdims=True)

    @pl.when(_first(pl.program_id(0)))
    def _():
        stats_ref[...] = jnp.zeros_like(stats_ref)

    stats_ref[...] += jnp.concatenate([s, ss], axis=0)


def _top_body(x_ref, wt_ref, b_ref, t_ref, stats_ref):
    t = jnp.dot(x_ref[...], wt_ref[...],
                preferred_element_type=jnp.float32) + b_ref[...]
    t_ref[...] = t
    _acc_stats(stats_ref, t)


def _qkv_body(t_ref, aphi_ref, cphi_ref, aqk_ref, cqk_ref, phi_ref, qk_ref):
    t = t_ref[...]
    phi_ref[...] = jnp.dot(t, aphi_ref[...],
                           preferred_element_type=jnp.float32) + cphi_ref[...]
    qk_ref[...] = jnp.dot(t, aqk_ref[...],
                          preferred_element_type=jnp.float32) + cqk_ref[...]


def _rep_rows(x):
    c = x.shape[-1]
    x3 = jnp.broadcast_to(x[:, None, :], (TN, KNB, c))
    return x3.reshape(TN * KNB, c)


def _rel(ppad_ref, pg_ref):
    return _rep_rows(ppad_ref[...]) - pg_ref[...]


def _relstats_body(ppad_ref, pg_ref, stats_ref):
    rel = _rel(ppad_ref, pg_ref)
    s = jnp.sum(rel, axis=0, keepdims=True)
    m2 = lax.dot_general(rel, rel, (((0,), (0,)), ((), ())),
                         preferred_element_type=jnp.float32)

    @pl.when(_first(pl.program_id(0)))
    def _():
        stats_ref[...] = jnp.zeros_like(stats_ref)

    stats_ref[...] += jnp.concatenate([s, m2], axis=0)


def _v_body(ppad_ref, pg_ref, w1_ref, c1_ref, w2_ref, b2_ref, v_ref, stats_ref):
    rel = _rel(ppad_ref, pg_ref)
    pe1 = jnp.maximum(
        jnp.dot(rel, w1_ref[...], preferred_element_type=jnp.float32)
        + c1_ref[...], 0.0)
    v = jnp.dot(pe1, w2_ref[...], preferred_element_type=jnp.float32) + b2_ref[...]
    v_ref[...] = v
    _acc_stats(stats_ref, v)


def _a1_body(phi_ref, pag_ref, v_ref, av_ref, cv_ref, wg1_ref, bg1_ref,
             a1_ref, stats_ref):
    pe = v_ref[...] * av_ref[...] + cv_ref[...]
    attn_in = _rep_rows(phi_ref[...]) - pag_ref[:, :D] + pe
    a1 = jnp.dot(attn_in, wg1_ref[...],
                 preferred_element_type=jnp.float32) + bg1_ref[...]
    a1_ref[...] = a1
    _acc_stats(stats_ref, a1)


def _w2_body(a1_ref, aa_ref, ca_ref, wg2_ref, bg2_ref, w2_ref, stats_ref):
    a2 = jnp.maximum(a1_ref[...] * aa_ref[...] + ca_ref[...], 0.0)
    w2 = jnp.dot(a2, wg2_ref[...],
                 preferred_element_type=jnp.float32) + bg2_ref[...]
    w2_ref[...] = w2
    _acc_stats(stats_ref, w2)


def _out_body(w2_ref, sc_ref, pag_ref, v_ref, av_ref, cv_ref,
              wd_ref, bd_ref, y2_ref, stats_ref):
    logits = (w2_ref[...] * sc_ref[...]).reshape(TN, KNB, D)
    m = jnp.max(logits, axis=1, keepdims=True)
    e = jnp.exp(logits - m)
    denom = jnp.sum(e, axis=1)
    val = (pag_ref[:, D:] + v_ref[...] * av_ref[...] + cv_ref[...]
           ).reshape(TN, KNB, D)
    y = jnp.sum(e * val, axis=1) / denom
    y2 = jnp.dot(y, wd_ref[...], preferred_element_type=jnp.float32) + bd_ref[...]
    y2_ref[...] = y2
    _acc_stats(stats_ref, y2)


def _res_body(y2_ref, ay_ref, cy_ref, x0_ref, out_ref):
    out_ref[...] = y2_ref[...] * ay_ref[...] + cy_ref[...] + x0_ref[...]


def _full(shape):
    return pl.BlockSpec(shape, lambda g: tuple(0 for _ in shape))


def _rows(tile, c):
    return pl.BlockSpec((tile, c), lambda g: (g, 0))


def _call(body, grid, in_specs, ins, out_specs, out_shapes):
    return pl.pallas_call(
        body, grid=grid, in_specs=in_specs,
        out_specs=out_specs, out_shape=out_shapes)(*ins)


def _fold_bn(mean, var, gamma, beta):
    inv = gamma / jnp.sqrt(var + EPS)
    return inv, beta - mean * inv


def _stats_to_musig(stats, m):
    mu = stats[0] / m
    var = stats[1] / m - mu * mu
    return mu, var


def kernel(input_p, input_x, params):
    b, _, n = input_p.shape
    r = b * n
    rk = r * KNB
    grid_pts = (r // TN,)

    p_rows = jnp.transpose(input_p, (0, 2, 1))
    x0 = jnp.transpose(input_x, (0, 2, 1)).reshape(r, D)
    p_pad = jnp.concatenate(
        [p_rows, jnp.zeros((b, n, PD - 3), jnp.float32)], axis=-1
    ).reshape(r, PD)

    idx = _knn(p_rows, input_p)
    idx_flat = idx.reshape(rk)

    p_g = _sc_gather(p_pad, idx_flat)

    t_full, t_stats = _call(
        _top_body, grid_pts,
        [_rows(TN, D), _full((D, D)), _full((1, D))],
        [x0, params['W_top'].T, params['b_top'][None, :]],
        [_rows(TN, D), _full((2, D))],
        [jax.ShapeDtypeStruct((r, D), jnp.float32),
         jax.ShapeDtypeStruct((2, D), jnp.float32)],
    )
    mu_t, var_t = _stats_to_musig(t_stats, r)
    inv_t, sh_t = _fold_bn(mu_t, var_t, params['g_top'], params['be_top'])
    def _fold_conv(w, bb):
        return inv_t[:, None] * w.T, (sh_t @ w.T + bb)[None, :]
    a_phi, c_phi = _fold_conv(params['W_phi'], params['b_phi'])
    a_psi, c_psi = _fold_conv(params['W_psi'], params['b_psi'])
    a_al, c_al = _fold_conv(params['W_alpha'], params['b_alpha'])
    a_qk = jnp.concatenate([a_psi, a_al], axis=1)
    c_qk = jnp.concatenate([c_psi, c_al], axis=1)

    phi, qk = _call(
        _qkv_body, grid_pts,
        [_rows(TN, D), _full((D, D)), _full((1, D)),
         _full((D, 2 * D)), _full((1, 2 * D))],
        [t_full, a_phi, c_phi, a_qk, c_qk],
        [_rows(TN, D), _rows(TN, 2 * D)],
        [jax.ShapeDtypeStruct((r, D), jnp.float32),
         jax.ShapeDtypeStruct((r, 2 * D), jnp.float32)],
    )

    pa_g = _sc_gather(qk, idx_flat)

    (rel_stats,) = _call(
        _relstats_body, grid_pts,
        [_rows(TN, PD), _rows(TRK, PD)],
        [p_pad, p_g],
        [_full((PD + 1, PD))],
        [jax.ShapeDtypeStruct((PD + 1, PD), jnp.float32)],
    )
    s_rel = rel_stats[0] / rk
    m2_rel = rel_stats[1:] / rk
    w_d1p = jnp.concatenate(
        [params['W_d1'], jnp.zeros((D, PD - 3), jnp.float32)], axis=1)
    mu_u = w_d1p @ s_rel + params['b_d1']
    e2_u = (jnp.einsum('oc,cd,od->o', w_d1p, m2_rel, w_d1p)
            + 2.0 * (w_d1p @ s_rel) * params['b_d1'] + params['b_d1'] ** 2)
    inv_u, sh_u = _fold_bn(mu_u, e2_u - mu_u * mu_u,
                           params['g_d1'], params['be_d1'])
    w1_hat = w_d1p.T * inv_u[None, :]
    c1_hat = (params['b_d1'] * inv_u + sh_u)[None, :]

    v_full, v_stats = _call(
        _v_body, grid_pts,
        [_rows(TN, PD), _rows(TRK, PD), _full((PD, D)), _full((1, D)),
         _full((D, D)), _full((1, D))],
        [p_pad, p_g, w1_hat, c1_hat, params['W_d2'].T,
         params['b_d2'][None, :]],
        [_rows(TRK, D), _full((2, D))],
        [jax.ShapeDtypeStruct((rk, D), jnp.float32),
         jax.ShapeDtypeStruct((2, D), jnp.float32)],
    )
    mu_v, var_v = _stats_to_musig(v_stats, rk)
    a_v, c_v = _fold_bn(mu_v, var_v, params['g_d2'], params['be_d2'])
    a_v, c_v = a_v[None, :], c_v[None, :]

    a1_full, a1_stats = _call(
        _a1_body, grid_pts,
        [_rows(TN, D), _rows(TRK, 2 * D), _rows(TRK, D), _full((1, D)),
         _full((1, D)), _full((D, D)), _full((1, D))],
        [phi, pa_g, v_full, a_v, c_v, params['W_g1'].T,
         params['b_g1'][None, :]],
        [_rows(TRK, D), _full((2, D))],
        [jax.ShapeDtypeStruct((rk, D), jnp.float32),
         jax.ShapeDtypeStruct((2, D), jnp.float32)],
    )
    mu_a1, var_a1 = _stats_to_musig(a1_stats, rk)
    a_a1, c_a1 = _fold_bn(mu_a1, var_a1, params['g_g1'], params['be_g1'])
    a_a1, c_a1 = a_a1[None, :], c_a1[None, :]

    w2_full, w2_stats = _call(
        _w2_body, grid_pts,
        [_rows(TRK, D), _full((1, D)), _full((1, D)), _full((D, D)),
         _full((1, D))],
        [a1_full, a_a1, c_a1, params['W_g2'].T, params['b_g2'][None, :]],
        [_rows(TRK, D), _full((2, D))],
        [jax.ShapeDtypeStruct((rk, D), jnp.float32),
         jax.ShapeDtypeStruct((2, D), jnp.float32)],
    )
    _, var_w2 = _stats_to_musig(w2_stats, rk)
    scale = (params['g_g2'] / jnp.sqrt(var_w2 + EPS))[None, :]

    y2_full, y2_stats = _call(
        _out_body, grid_pts,
        [_rows(TRK, D), _full((1, D)), _rows(TRK, 2 * D), _rows(TRK, D),
         _full((1, D)), _full((1, D)), _full((D, D)), _full((1, D))],
        [w2_full, scale, pa_g, v_full, a_v, c_v, params['W_down'].T,
         params['b_down'][None, :]],
        [_rows(TN, D), _full((2, D))],
        [jax.ShapeDtypeStruct((r, D), jnp.float32),
         jax.ShapeDtypeStruct((2, D), jnp.float32)],
    )
    mu_y2, var_y2 = _stats_to_musig(y2_stats, r)
    a_y, c_y = _fold_bn(mu_y2, var_y2, params['g_down'], params['be_down'])

    (out_rows,) = _call(
        _res_body, grid_pts,
        [_rows(TN, D), _full((1, D)), _full((1, D)), _rows(TN, D)],
        [y2_full, a_y[None, :], c_y[None, :], x0],
        [_rows(TN, D)],
        [jax.ShapeDtypeStruct((r, D), jnp.float32)],
    )
    return jnp.transpose(out_rows.reshape(b, n, D), (0, 2, 1))


# speedup vs baseline: 16.4633x; 1.0431x over previous
import functools

import jax
import jax.numpy as jnp
from jax import lax
from jax.experimental import pallas as pl
from jax.experimental.pallas import tpu as pltpu
from jax.experimental.pallas import tpu_sc as plsc

D = 64
KNB = 16
PD = 16
EPS = 1e-5

TR = 512
TN = 1024
TRK = TN * KNB

NC = 2
NS = 16
NW = NC * NS
CH = 128


def _knn_body(prow_ref, pcol_ref, idx_ref, *, n):
    pr = prow_ref[0]
    pc = pcol_ref[0]
    sq_r = jnp.zeros((TR, 1), jnp.float32)
    sq_c = jnp.zeros((1, n), jnp.float32)
    dot = jnp.zeros((TR, n), jnp.float32)
    for d in range(3):
        prd = pr[:, d:d + 1]
        pcd = pc[d:d + 1, :]
        sq_r = sq_r + prd * prd
        sq_c = sq_c + pcd * pcd
        prb = prd.astype(jnp.bfloat16).astype(jnp.float32)
        pcb = pcd.astype(jnp.bfloat16).astype(jnp.float32)
        dot = dot + prb * pcb
    acc = sq_r + sq_c - 2.0 * dot
    lanes = lax.broadcasted_iota(jnp.int32, (TR, n), 1)
    off = pl.program_id(0) * n
    for t in range(KNB):
        am = jnp.argmin(acc, axis=1).astype(jnp.int32)[:, None]
        idx_ref[0, :, t:t + 1] = am + off
        acc = jnp.where(lanes == am, jnp.float32(jnp.inf), acc)


def _knn(p_rows, p_cols):
    b, n, _ = p_rows.shape
    return pl.pallas_call(
        functools.partial(_knn_body, n=n),
        grid=(b, n // TR),
        in_specs=[
            pl.BlockSpec((1, TR, 3), lambda bb, i: (bb, i, 0)),
            pl.BlockSpec((1, 3, n), lambda bb, i: (bb, 0, 0)),
        ],
        out_specs=pl.BlockSpec((1, TR, KNB), lambda bb, i: (bb, i, 0)),
        out_shape=jax.ShapeDtypeStruct((b, n, KNB), jnp.int32),
    )(p_rows, p_cols)


def _sc_gather(tab, idx_flat):
    rk = idx_flat.shape[0]
    c = tab.shape[1]
    per_w = rk // NW
    mesh = plsc.VectorSubcoreMesh(core_axis_name="c", subcore_axis_name="s")

    @functools.partial(
        pl.kernel,
        mesh=mesh,
        compiler_params=pltpu.CompilerParams(use_tc_tiling_on_sc=False),
        out_type=jax.ShapeDtypeStruct((rk, c), jnp.float32),
        scratch_types=[
            pltpu.VMEM((per_w,), jnp.int32),
            pltpu.VMEM((CH, c), jnp.float32),
            pltpu.VMEM((CH, c), jnp.float32),
            pltpu.SemaphoreType.DMA,
            pltpu.SemaphoreType.DMA,
            pltpu.SemaphoreType.DMA,
            pltpu.SemaphoreType.DMA,
        ],
    )
    def k(tab_hbm, idx_hbm, out_hbm, idx_all, rows_v0, rows_v1,
          semg0, semg1, semo0, semo1):
        wid = lax.axis_index("s") * NC + lax.axis_index("c")
        base = wid * per_w
        pltpu.sync_copy(idx_hbm.at[pl.ds(base, per_w)], idx_all)

        def body(i, carry):
            g0 = 2 * i
            g1 = 2 * i + 1
            cg0 = pltpu.async_copy(
                tab_hbm.at[idx_all.at[pl.ds(g0 * CH, CH)]], rows_v0, semg0)
            cg1 = pltpu.async_copy(
                tab_hbm.at[idx_all.at[pl.ds(g1 * CH, CH)]], rows_v1, semg1)
            cg0.wait()
            co0 = pltpu.async_copy(
                rows_v0, out_hbm.at[pl.ds(base + g0 * CH, CH)], semo0)
            cg1.wait()
            co1 = pltpu.async_copy(
                rows_v1, out_hbm.at[pl.ds(base + g1 * CH, CH)], semo1)
            co0.wait()
            co1.wait()
            return carry

        lax.fori_loop(0, per_w // (2 * CH), body, 0)

    return k(tab, idx_flat)


def _first(g):
    return g == 0


def _acc_stats(stats_ref, vals):
    s = jnp.sum(vals, axis=0, keepdims=True)
    ss = jnp.sum(vals * vals, axis=0, keepdims=True)

    @pl.when(_first(pl.program_id(0)))
    def _():
        stats_ref[...] = jnp.zeros_like(stats_ref)

    stats_ref[...] += jnp.concatenate([s, ss], axis=0)


def _top_body(x_ref, wt_ref, b_ref, t_ref, stats_ref):
    t = jnp.dot(x_ref[...], wt_ref[...],
                preferred_element_type=jnp.float32) + b_ref[...]
    t_ref[...] = t
    _acc_stats(stats_ref, t)


def _qkv_body(t_ref, aphi_ref, cphi_ref, aqk_ref, cqk_ref, phi_ref, qk_ref):
    t = t_ref[...]
    phi_ref[...] = jnp.dot(t, aphi_ref[...],
                           preferred_element_type=jnp.float32) + cphi_ref[...]
    qk_ref[...] = jnp.dot(t, aqk_ref[...],
                          preferred_element_type=jnp.float32) + cqk_ref[...]


def _rep_rows(x):
    c = x.shape[-1]
    x3 = jnp.broadcast_to(x[:, None, :], (TN, KNB, c))
    return x3.reshape(TN * KNB, c)


def _rel(ppad_ref, pg_ref):
    return _rep_rows(ppad_ref[...]) - pg_ref[...]


def _relstats_body(ppad_ref, pg_ref, stats_ref):
    rel = _rel(ppad_ref, pg_ref)
    s = jnp.sum(rel, axis=0, keepdims=True)
    m2 = lax.dot_general(rel, rel, (((0,), (0,)), ((), ())),
                         preferred_element_type=jnp.float32)

    @pl.when(_first(pl.program_id(0)))
    def _():
        stats_ref[...] = jnp.zeros_like(stats_ref)

    stats_ref[...] += jnp.concatenate([s, m2], axis=0)


def _v_body(ppad_ref, pg_ref, w1_ref, c1_ref, w2_ref, b2_ref, v_ref, stats_ref):
    rel = _rel(ppad_ref, pg_ref)
    pe1 = jnp.maximum(
        jnp.dot(rel, w1_ref[...], preferred_element_type=jnp.float32)
        + c1_ref[...], 0.0)
    v = jnp.dot(pe1, w2_ref[...], preferred_element_type=jnp.float32) + b2_ref[...]
    v_ref[...] = v
    _acc_stats(stats_ref, v)


def _a1_body(phi_ref, pag_ref, v_ref, av_ref, cv_ref, wg1_ref, bg1_ref,
             a1_ref, stats_ref):
    pe = v_ref[...] * av_ref[...] + cv_ref[...]
    attn_in = _rep_rows(phi_ref[...]) - pag_ref[:, :D] + pe
    a1 = jnp.dot(attn_in, wg1_ref[...],
                 preferred_element_type=jnp.float32) + bg1_ref[...]
    a1_ref[...] = a1
    _acc_stats(stats_ref, a1)


def _w2_body(a1_ref, aa_ref, ca_ref, wg2_ref, bg2_ref, w2_ref, stats_ref):
    a2 = jnp.maximum(a1_ref[...] * aa_ref[...] + ca_ref[...], 0.0)
    w2 = jnp.dot(a2, wg2_ref[...],
                 preferred_element_type=jnp.float32) + bg2_ref[...]
    w2_ref[...] = w2
    _acc_stats(stats_ref, w2)


def _out_body(w2_ref, sc_ref, pag_ref, v_ref, av_ref, cv_ref,
              wd_ref, bd_ref, y2_ref, stats_ref):
    logits = (w2_ref[...] * sc_ref[...]).reshape(TN, KNB, D)
    m = jnp.max(logits, axis=1, keepdims=True)
    e = jnp.exp(logits - m)
    denom = jnp.sum(e, axis=1)
    val = (pag_ref[:, D:] + v_ref[...] * av_ref[...] + cv_ref[...]
           ).reshape(TN, KNB, D)
    y = jnp.sum(e * val, axis=1) / denom
    y2 = jnp.dot(y, wd_ref[...], preferred_element_type=jnp.float32) + bd_ref[...]
    y2_ref[...] = y2
    _acc_stats(stats_ref, y2)


def _res_body(y2_ref, ay_ref, cy_ref, x0_ref, out_ref):
    out_ref[...] = y2_ref[...] * ay_ref[...] + cy_ref[...] + x0_ref[...]


def _full(shape):
    return pl.BlockSpec(shape, lambda g: tuple(0 for _ in shape))


def _rows(tile, c):
    return pl.BlockSpec((tile, c), lambda g: (g, 0))


def _call(body, grid, in_specs, ins, out_specs, out_shapes):
    return pl.pallas_call(
        body, grid=grid, in_specs=in_specs,
        out_specs=out_specs, out_shape=out_shapes)(*ins)


def _fold_bn(mean, var, gamma, beta):
    inv = gamma / jnp.sqrt(var + EPS)
    return inv, beta - mean * inv


def _stats_to_musig(stats, m):
    mu = stats[0] / m
    var = stats[1] / m - mu * mu
    return mu, var


def kernel(input_p, input_x, params):
    b, _, n = input_p.shape
    r = b * n
    rk = r * KNB
    grid_pts = (r // TN,)

    p_rows = jnp.transpose(input_p, (0, 2, 1))
    x0 = jnp.transpose(input_x, (0, 2, 1)).reshape(r, D)
    p_pad = jnp.concatenate(
        [p_rows, jnp.zeros((b, n, PD - 3), jnp.float32)], axis=-1
    ).reshape(r, PD)

    idx = _knn(p_rows, input_p)
    idx_flat = idx.reshape(rk)

    p_g = _sc_gather(p_pad, idx_flat)

    t_full, t_stats = _call(
        _top_body, grid_pts,
        [_rows(TN, D), _full((D, D)), _full((1, D))],
        [x0, params['W_top'].T, params['b_top'][None, :]],
        [_rows(TN, D), _full((2, D))],
        [jax.ShapeDtypeStruct((r, D), jnp.float32),
         jax.ShapeDtypeStruct((2, D), jnp.float32)],
    )
    mu_t, var_t = _stats_to_musig(t_stats, r)
    inv_t, sh_t = _fold_bn(mu_t, var_t, params['g_top'], params['be_top'])
    def _fold_conv(w, bb):
        return inv_t[:, None] * w.T, (sh_t @ w.T + bb)[None, :]
    a_phi, c_phi = _fold_conv(params['W_phi'], params['b_phi'])
    a_psi, c_psi = _fold_conv(params['W_psi'], params['b_psi'])
    a_al, c_al = _fold_conv(params['W_alpha'], params['b_alpha'])
    a_qk = jnp.concatenate([a_psi, a_al], axis=1)
    c_qk = jnp.concatenate([c_psi, c_al], axis=1)

    phi, qk = _call(
        _qkv_body, grid_pts,
        [_rows(TN, D), _full((D, D)), _full((1, D)),
         _full((D, 2 * D)), _full((1, 2 * D))],
        [t_full, a_phi, c_phi, a_qk, c_qk],
        [_rows(TN, D), _rows(TN, 2 * D)],
        [jax.ShapeDtypeStruct((r, D), jnp.float32),
         jax.ShapeDtypeStruct((r, 2 * D), jnp.float32)],
    )

    pa_g = _sc_gather(qk, idx_flat)

    (rel_stats,) = _call(
        _relstats_body, grid_pts,
        [_rows(TN, PD), _rows(TRK, PD)],
        [p_pad, p_g],
        [_full((PD + 1, PD))],
        [jax.ShapeDtypeStruct((PD + 1, PD), jnp.float32)],
    )
    s_rel = rel_stats[0] / rk
    m2_rel = rel_stats[1:] / rk
    w_d1p = jnp.concatenate(
        [params['W_d1'], jnp.zeros((D, PD - 3), jnp.float32)], axis=1)
    mu_u = w_d1p @ s_rel + params['b_d1']
    e2_u = (jnp.einsum('oc,cd,od->o', w_d1p, m2_rel, w_d1p)
            + 2.0 * (w_d1p @ s_rel) * params['b_d1'] + params['b_d1'] ** 2)
    inv_u, sh_u = _fold_bn(mu_u, e2_u - mu_u * mu_u,
                           params['g_d1'], params['be_d1'])
    w1_hat = w_d1p.T * inv_u[None, :]
    c1_hat = (params['b_d1'] * inv_u + sh_u)[None, :]

    v_full, v_stats = _call(
        _v_body, grid_pts,
        [_rows(TN, PD), _rows(TRK, PD), _full((PD, D)), _full((1, D)),
         _full((D, D)), _full((1, D))],
        [p_pad, p_g, w1_hat, c1_hat, params['W_d2'].T,
         params['b_d2'][None, :]],
        [_rows(TRK, D), _full((2, D))],
        [jax.ShapeDtypeStruct((rk, D), jnp.float32),
         jax.ShapeDtypeStruct((2, D), jnp.float32)],
    )
    mu_v, var_v = _stats_to_musig(v_stats, rk)
    a_v, c_v = _fold_bn(mu_v, var_v, params['g_d2'], params['be_d2'])
    a_v, c_v = a_v[None, :], c_v[None, :]

    a1_full, a1_stats = _call(
        _a1_body, grid_pts,
        [_rows(TN, D), _rows(TRK, 2 * D), _rows(TRK, D), _full((1, D)),
         _full((1, D)), _full((D, D)), _full((1, D))],
        [phi, pa_g, v_full, a_v, c_v, params['W_g1'].T,
         params['b_g1'][None, :]],
        [_rows(TRK, D), _full((2, D))],
        [jax.ShapeDtypeStruct((rk, D), jnp.float32),
         jax.ShapeDtypeStruct((2, D), jnp.float32)],
    )
    mu_a1, var_a1 = _stats_to_musig(a1_stats, rk)
    a_a1, c_a1 = _fold_bn(mu_a1, var_a1, params['g_g1'], params['be_g1'])
    a_a1, c_a1 = a_a1[None, :], c_a1[None, :]

    w2_full, w2_stats = _call(
        _w2_body, grid_pts,
        [_rows(TRK, D), _full((1, D)), _full((1, D)), _full((D, D)),
         _full((1, D))],
        [a1_full, a_a1, c_a1, params['W_g2'].T, params['b_g2'][None, :]],
        [_rows(TRK, D), _full((2, D))],
        [jax.ShapeDtypeStruct((rk, D), jnp.float32),
         jax.ShapeDtypeStruct((2, D), jnp.float32)],
    )
    _, var_w2 = _stats_to_musig(w2_stats, rk)
    scale = (params['g_g2'] / jnp.sqrt(var_w2 + EPS))[None, :]

    y2_full, y2_stats = _call(
        _out_body, grid_pts,
        [_rows(TRK, D), _full((1, D)), _rows(TRK, 2 * D), _rows(TRK, D),
         _full((1, D)), _full((1, D)), _full((D, D)), _full((1, D))],
        [w2_full, scale, pa_g, v_full, a_v, c_v, params['W_down'].T,
         params['b_down'][None, :]],
        [_rows(TN, D), _full((2, D))],
        [jax.ShapeDtypeStruct((r, D), jnp.float32),
         jax.ShapeDtypeStruct((2, D), jnp.float32)],
    )
    mu_y2, var_y2 = _stats_to_musig(y2_stats, r)
    a_y, c_y = _fold_bn(mu_y2, var_y2, params['g_down'], params['be_down'])

    (out_rows,) = _call(
        _res_body, grid_pts,
        [_rows(TN, D), _full((1, D)), _full((1, D)), _rows(TN, D)],
        [y2_full, a_y[None, :], c_y[None, :], x0],
        [_rows(TN, D)],
        [jax.ShapeDtypeStruct((r, D), jnp.float32)],
    )
    return jnp.transpose(out_rows.reshape(b, n, D), (0, 2, 1))
